# baseline (speedup 1.0000x reference)
import jax
import jax.numpy as jnp
from jax.experimental import pallas as pl

N = 4096
BLK_R = 4096
BLK_C = 512


def _flip_block(x_ref, p_ref, o_ref):
    o_ref[...] = jax.lax.dot(
        x_ref[...], p_ref[...], preferred_element_type=jnp.float32
    )


def kernel(inputs):
    nr = N // BLK_R
    nc = N // BLK_C
    rev = jnp.equal(
        jnp.arange(BLK_C)[:, None] + jnp.arange(BLK_C)[None, :], BLK_C - 1
    ).astype(jnp.float32)
    return pl.pallas_call(
        _flip_block,
        grid=(nr, nc),
        in_specs=[
            pl.BlockSpec((BLK_R, BLK_C), lambda i, j: (i, nc - 1 - j)),
            pl.BlockSpec((BLK_C, BLK_C), lambda i, j: (0, 0)),
        ],
        out_specs=pl.BlockSpec((BLK_R, BLK_C), lambda i, j: (i, j)),
        out_shape=jax.ShapeDtypeStruct((N, N), jnp.float32),
    )(inputs, rev)


# speedup vs baseline: 1.0587x; 1.0587x over previous
import jax
import jax.numpy as jnp
from jax.experimental import pallas as pl

N = 4096
BLK_R = 4096
BLK_C = 512
STRIP = 128


def _flip_block(x_ref, p_ref, o_ref):
    ns = BLK_C // STRIP
    p = p_ref[...]
    for s in range(ns):
        src = (ns - 1 - s) * STRIP
        o_ref[:, s * STRIP:(s + 1) * STRIP] = jax.lax.dot(
            x_ref[:, src:src + STRIP], p, preferred_element_type=jnp.float32
        )


def kernel(inputs):
    nr = N // BLK_R
    nc = N // BLK_C
    rev = jnp.equal(
        jnp.arange(STRIP)[:, None] + jnp.arange(STRIP)[None, :], STRIP - 1
    ).astype(jnp.float32)
    return pl.pallas_call(
        _flip_block,
        grid=(nr, nc),
        in_specs=[
            pl.BlockSpec((BLK_R, BLK_C), lambda i, j: (i, nc - 1 - j)),
            pl.BlockSpec((STRIP, STRIP), lambda i, j: (0, 0)),
        ],
        out_specs=pl.BlockSpec((BLK_R, BLK_C), lambda i, j: (i, j)),
        out_shape=jax.ShapeDtypeStruct((N, N), jnp.float32),
    )(inputs, rev)


# speedup vs baseline: 1.0677x; 1.0086x over previous
import jax
import jax.numpy as jnp
from jax.experimental import pallas as pl

N = 4096
BLK_R = 512
STRIP = 128


def _flip_block(x_ref, p_ref, o_ref):
    ns = N // STRIP
    p = p_ref[...]
    for s in range(ns):
        src = (ns - 1 - s) * STRIP
        o_ref[:, s * STRIP:(s + 1) * STRIP] = jax.lax.dot(
            x_ref[:, src:src + STRIP], p, preferred_element_type=jnp.float32
        )


def kernel(inputs):
    nr = N // BLK_R
    rev = jnp.equal(
        jnp.arange(STRIP)[:, None] + jnp.arange(STRIP)[None, :], STRIP - 1
    ).astype(jnp.float32)
    return pl.pallas_call(
        _flip_block,
        grid=(nr,),
        in_specs=[
            pl.BlockSpec((BLK_R, N), lambda i: (i, 0)),
            pl.BlockSpec((STRIP, STRIP), lambda i: (0, 0)),
        ],
        out_specs=pl.BlockSpec((BLK_R, N), lambda i: (i, 0)),
        out_shape=jax.ShapeDtypeStruct((N, N), jnp.float32),
    )(inputs, rev)
